# baseline (speedup 1.0000x reference)
import functools

import jax
import jax.numpy as jnp
from jax import lax
from jax.experimental import pallas as pl
from jax.experimental.pallas import tpu as pltpu
from jax.experimental.pallas import tpu_sc as plsc

N = 10000
E = 320000
NODE_DIM = 128
EDGE_DIM = 16
OUT_DIM = 128
CDIM = OUT_DIM + 16

EBLK = 2560

NW = 32
GCH = 400
GCHUNKS = E // (NW * GCH)


def _make_gather():
    mesh = plsc.VectorSubcoreMesh(core_axis_name="c", subcore_axis_name="s")

    @functools.partial(
        pl.kernel,
        out_type=jax.ShapeDtypeStruct((E, OUT_DIM), jnp.float32),
        mesh=mesh,
        scratch_types=[
            pltpu.VMEM((GCH,), jnp.int32),
            pltpu.VMEM((GCH, OUT_DIM), jnp.float32),
            pltpu.SemaphoreType.DMA,
        ],
    )
    def gather_k(xa_hbm, dst_hbm, out_hbm, idx_v, rows_v, sem):
        wid = lax.axis_index("s") * 2 + lax.axis_index("c")
        base = wid * (E // NW)
        for c in range(GCHUNKS):
            off = base + c * GCH
            pltpu.sync_copy(dst_hbm.at[pl.ds(off, GCH)], idx_v)
            pltpu.async_copy(xa_hbm.at[idx_v], rows_v, sem).wait()
            pltpu.sync_copy(rows_v, out_hbm.at[pl.ds(off, GCH)])

    return gather_k


_gather = _make_gather()


def _xa_kernel(x_ref, w_ref, b_ref, out_ref):
    out_ref[...] = jnp.dot(x_ref[...], w_ref[...],
                           preferred_element_type=jnp.float32) + b_ref[...]


def _msg_kernel(g_ref, ea_ref, w1b_ref, w2_ref, b2_ref, out_ref):
    h = g_ref[...] + jnp.dot(ea_ref[...], w1b_ref[...],
                             preferred_element_type=jnp.float32)
    h = jnp.maximum(h, 0.0)
    msg = jnp.dot(h, w2_ref[...],
                  preferred_element_type=jnp.float32) + b2_ref[...]
    pad = jax.lax.broadcasted_iota(jnp.int32, (EBLK, CDIM - OUT_DIM), 1)
    out_ref[...] = jnp.concatenate(
        [msg, jnp.where(pad == 0, 1.0, 0.0)], axis=1)


def _combine_kernel(sum_ref, max_ref, cnt_ref, wmean_ref, wmax_ref,
                    wsum_ref, b_ref, out_ref):
    s = sum_ref[...]
    cnt = cnt_ref[...]
    mean = s / jnp.maximum(cnt, 1.0)
    mx = jnp.where(cnt > 0.0, max_ref[...], 0.0)
    out = jnp.dot(mean, wmean_ref[...], preferred_element_type=jnp.float32)
    out += jnp.dot(mx, wmax_ref[...], preferred_element_type=jnp.float32)
    out += jnp.dot(s, wsum_ref[...], preferred_element_type=jnp.float32)
    out_ref[...] = out + b_ref[...]


@jax.jit
def kernel(x, edge_index, edge_attr, W1, b1, W2, b2, Wmean, bmean, Wmax, bmax,
           Wsum, bsum):
    dst = edge_index[1]

    xa = pl.pallas_call(
        _xa_kernel,
        out_shape=jax.ShapeDtypeStruct((N, OUT_DIM), jnp.float32),
    )(x, W1[:NODE_DIM], b1.reshape(1, OUT_DIM))

    g = _gather(xa, dst)

    msg144 = pl.pallas_call(
        _msg_kernel,
        grid=(E // EBLK,),
        in_specs=[
            pl.BlockSpec((EBLK, OUT_DIM), lambda i: (i, 0)),
            pl.BlockSpec((EBLK, EDGE_DIM), lambda i: (i, 0)),
            pl.BlockSpec((EDGE_DIM, OUT_DIM), lambda i: (0, 0)),
            pl.BlockSpec((OUT_DIM, OUT_DIM), lambda i: (0, 0)),
            pl.BlockSpec((1, OUT_DIM), lambda i: (0, 0)),
        ],
        out_specs=pl.BlockSpec((EBLK, CDIM), lambda i: (i, 0)),
        out_shape=jax.ShapeDtypeStruct((E, CDIM), jnp.float32),
    )(g, edge_attr, W1[NODE_DIM:], W2, b2.reshape(1, OUT_DIM))

    sumc = jax.ops.segment_sum(msg144, dst, num_segments=N)
    maxc = jax.ops.segment_max(msg144, dst, num_segments=N)

    out = pl.pallas_call(
        _combine_kernel,
        out_shape=jax.ShapeDtypeStruct((N, OUT_DIM), jnp.float32),
    )(sumc[:, :OUT_DIM], maxc[:, :OUT_DIM], sumc[:, OUT_DIM:OUT_DIM + 1],
      Wmean, Wmax, Wsum, (bmean + bmax + bsum).reshape(1, OUT_DIM))
    return out


# speedup vs baseline: 1.0354x; 1.0354x over previous
import functools

import jax
import jax.numpy as jnp
from jax import lax
from jax.experimental import pallas as pl
from jax.experimental.pallas import tpu as pltpu
from jax.experimental.pallas import tpu_sc as plsc

N = 10000
E = 320000
NODE_DIM = 128
EDGE_DIM = 16
OUT_DIM = 128
CDIM = OUT_DIM + 16

EBLK = 2560

NW = 32
GCH = 400
GCHUNKS = E // (NW * GCH)


def _make_gather():
    mesh = plsc.VectorSubcoreMesh(core_axis_name="c", subcore_axis_name="s")

    @functools.partial(
        pl.kernel,
        out_type=jax.ShapeDtypeStruct((E, OUT_DIM), jnp.float32),
        mesh=mesh,
        scratch_types=[
            pltpu.VMEM((GCH,), jnp.int32),
            pltpu.VMEM((GCH, OUT_DIM), jnp.float32),
            pltpu.SemaphoreType.DMA,
        ],
    )
    def gather_k(xa_hbm, dst_hbm, out_hbm, idx_v, rows_v, sem):
        wid = lax.axis_index("s") * 2 + lax.axis_index("c")
        base = wid * (E // NW)
        for c in range(GCHUNKS):
            off = base + c * GCH
            pltpu.sync_copy(dst_hbm.at[pl.ds(off, GCH)], idx_v)
            pltpu.async_copy(xa_hbm.at[idx_v], rows_v, sem).wait()
            pltpu.sync_copy(rows_v, out_hbm.at[pl.ds(off, GCH)])

    return gather_k


_gather = _make_gather()


def _xa_kernel(x_ref, w_ref, b_ref, out_ref):
    out_ref[...] = jnp.dot(x_ref[...], w_ref[...],
                           preferred_element_type=jnp.float32) + b_ref[...]


def _msg_kernel(g_ref, ea_ref, w1b_ref, w2_ref, b2_ref, out_ref):
    h = g_ref[...] + jnp.dot(ea_ref[...], w1b_ref[...],
                             preferred_element_type=jnp.float32)
    h = jnp.maximum(h, 0.0)
    out_ref[...] = jnp.dot(h, w2_ref[...],
                           preferred_element_type=jnp.float32) + b2_ref[...]


def _combine_kernel(sum_ref, max_ref, cnt_ref, wmean_ref, wmax_ref,
                    wsum_ref, b_ref, out_ref):
    s = sum_ref[...]
    cnt = cnt_ref[...]
    mean = s / jnp.maximum(cnt, 1.0)
    mx = jnp.where(cnt > 0.0, max_ref[...], 0.0)
    out = jnp.dot(mean, wmean_ref[...], preferred_element_type=jnp.float32)
    out += jnp.dot(mx, wmax_ref[...], preferred_element_type=jnp.float32)
    out += jnp.dot(s, wsum_ref[...], preferred_element_type=jnp.float32)
    out_ref[...] = out + b_ref[...]


@jax.jit
def kernel(x, edge_index, edge_attr, W1, b1, W2, b2, Wmean, bmean, Wmax, bmax,
           Wsum, bsum):
    dst = edge_index[1]

    xa = pl.pallas_call(
        _xa_kernel,
        out_shape=jax.ShapeDtypeStruct((N, OUT_DIM), jnp.float32),
    )(x, W1[:NODE_DIM], b1.reshape(1, OUT_DIM))

    g = _gather(xa, dst)

    msg144 = pl.pallas_call(
        _msg_kernel,
        grid=(E // EBLK,),
        in_specs=[
            pl.BlockSpec((EBLK, OUT_DIM), lambda i: (i, 0)),
            pl.BlockSpec((EBLK, EDGE_DIM), lambda i: (i, 0)),
            pl.BlockSpec((EDGE_DIM, OUT_DIM), lambda i: (0, 0)),
            pl.BlockSpec((OUT_DIM, OUT_DIM), lambda i: (0, 0)),
            pl.BlockSpec((1, OUT_DIM), lambda i: (0, 0)),
        ],
        out_specs=pl.BlockSpec((EBLK, OUT_DIM), lambda i: (i, 0)),
        out_shape=jax.ShapeDtypeStruct((E, OUT_DIM), jnp.float32),
    )(g, edge_attr, W1[NODE_DIM:], W2, b2.reshape(1, OUT_DIM))

    sum_agg = jax.ops.segment_sum(msg144, dst, num_segments=N)
    cnt = jax.ops.segment_sum(jnp.ones((E,), jnp.float32), dst, num_segments=N)
    max_agg = jax.ops.segment_max(msg144, dst, num_segments=N)

    out = pl.pallas_call(
        _combine_kernel,
        out_shape=jax.ShapeDtypeStruct((N, OUT_DIM), jnp.float32),
    )(sum_agg, max_agg, cnt.reshape(N, 1),
      Wmean, Wmax, Wsum, (bmean + bmax + bsum).reshape(1, OUT_DIM))
    return out


# speedup vs baseline: 1.1094x; 1.0714x over previous
import functools

import jax
import jax.numpy as jnp
from jax import lax
from jax.experimental import pallas as pl
from jax.experimental.pallas import tpu as pltpu
from jax.experimental.pallas import tpu_sc as plsc

N = 10000
E = 320000
NODE_DIM = 128
EDGE_DIM = 16
OUT_DIM = 128
CDIM = OUT_DIM + 16

EBLK = 2560

NW = 32
GCH = 400
GCHUNKS = E // (NW * GCH)


def _make_gather():
    mesh = plsc.VectorSubcoreMesh(core_axis_name="c", subcore_axis_name="s")

    @functools.partial(
        pl.kernel,
        out_type=jax.ShapeDtypeStruct((E, OUT_DIM), jnp.float32),
        mesh=mesh,
        scratch_types=[
            pltpu.VMEM((GCH,), jnp.int32),
            pltpu.VMEM((GCH, OUT_DIM), jnp.float32),
            pltpu.SemaphoreType.DMA,
        ],
    )
    def gather_k(xa_hbm, dst_hbm, out_hbm, idx_v, rows_v, sem):
        wid = lax.axis_index("s") * 2 + lax.axis_index("c")
        base = wid * (E // NW)
        for c in range(GCHUNKS):
            off = base + c * GCH
            pltpu.sync_copy(dst_hbm.at[pl.ds(off, GCH)], idx_v)
            pltpu.async_copy(xa_hbm.at[idx_v], rows_v, sem).wait()
            pltpu.sync_copy(rows_v, out_hbm.at[pl.ds(off, GCH)])

    return gather_k


_gather = _make_gather()


def _xa_kernel(x_ref, w_ref, b_ref, out_ref):
    out_ref[...] = jnp.dot(x_ref[...], w_ref[...],
                           preferred_element_type=jnp.float32) + b_ref[...]


def _msg_kernel(g_ref, ea_ref, w1b_ref, w2_ref, b2_ref, out_ref, outbf_ref):
    h = g_ref[...] + jnp.dot(ea_ref[...], w1b_ref[...],
                             preferred_element_type=jnp.float32)
    h = jnp.maximum(h, 0.0)
    msg = jnp.dot(h, w2_ref[...],
                  preferred_element_type=jnp.float32) + b2_ref[...]
    out_ref[...] = msg
    outbf_ref[...] = msg.astype(jnp.bfloat16)


def _combine_kernel(sum_ref, max_ref, cnt_ref, wmean_ref, wmax_ref,
                    wsum_ref, b_ref, out_ref):
    s = sum_ref[...]
    cnt = cnt_ref[...]
    mean = s / jnp.maximum(cnt, 1.0)
    mx = jnp.where(cnt > 0.0, max_ref[...], 0.0)
    out = jnp.dot(mean, wmean_ref[...], preferred_element_type=jnp.float32)
    out += jnp.dot(mx, wmax_ref[...], preferred_element_type=jnp.float32)
    out += jnp.dot(s, wsum_ref[...], preferred_element_type=jnp.float32)
    out_ref[...] = out + b_ref[...]


@jax.jit
def kernel(x, edge_index, edge_attr, W1, b1, W2, b2, Wmean, bmean, Wmax, bmax,
           Wsum, bsum):
    dst = edge_index[1]

    xa = pl.pallas_call(
        _xa_kernel,
        out_shape=jax.ShapeDtypeStruct((N, OUT_DIM), jnp.float32),
    )(x, W1[:NODE_DIM], b1.reshape(1, OUT_DIM))

    g = _gather(xa, dst)

    msg144 = pl.pallas_call(
        _msg_kernel,
        grid=(E // EBLK,),
        in_specs=[
            pl.BlockSpec((EBLK, OUT_DIM), lambda i: (i, 0)),
            pl.BlockSpec((EBLK, EDGE_DIM), lambda i: (i, 0)),
            pl.BlockSpec((EDGE_DIM, OUT_DIM), lambda i: (0, 0)),
            pl.BlockSpec((OUT_DIM, OUT_DIM), lambda i: (0, 0)),
            pl.BlockSpec((1, OUT_DIM), lambda i: (0, 0)),
        ],
        out_specs=[pl.BlockSpec((EBLK, OUT_DIM), lambda i: (i, 0)),
                   pl.BlockSpec((EBLK, OUT_DIM), lambda i: (i, 0))],
        out_shape=[jax.ShapeDtypeStruct((E, OUT_DIM), jnp.float32),
                   jax.ShapeDtypeStruct((E, OUT_DIM), jnp.bfloat16)],
    )(g, edge_attr, W1[NODE_DIM:], W2, b2.reshape(1, OUT_DIM))
    msg, msg_bf = msg144

    sum_agg = jax.ops.segment_sum(msg, dst, num_segments=N)
    cnt = jax.ops.segment_sum(jnp.ones((E,), jnp.float32), dst, num_segments=N)
    max_agg = jax.ops.segment_max(msg_bf, dst,
                                  num_segments=N).astype(jnp.float32)

    out = pl.pallas_call(
        _combine_kernel,
        out_shape=jax.ShapeDtypeStruct((N, OUT_DIM), jnp.float32),
    )(sum_agg, max_agg, cnt.reshape(N, 1),
      Wmean, Wmax, Wsum, (bmean + bmax + bsum).reshape(1, OUT_DIM))
    return out
